# baseline (speedup 1.0000x reference)
import jax
import jax.numpy as jnp
from jax import lax
from jax.experimental import pallas as pl
from jax.experimental.pallas import tpu as pltpu
from jax.experimental.pallas import tpu_sc as plsc

B, K, H, W = 1024, 3, 8, 8
HW = H * W
D = 128
NC, NS, L = 2, 16, 16
NW = NC * NS
BPW = B // NW
IPU = 2
UPW = BPW // IPU
UHW = IPU * HW
CD = D // L
CH = HW // L


def _sc_body(inp_hbm, table_hbm, out_hbm,
             idx_v, acc0, acc1, accT0, accT1,
             gsem0, gsem1, osem0, osem1):
    wid = lax.axis_index("s") * NC + lax.axis_index("c")
    u0 = wid * UPW
    pltpu.sync_copy(inp_hbm.at[pl.ds(u0 * K, UPW * K)], idx_v)

    lanes = lax.iota(jnp.int32, L)
    row_vecs = [[m * HW + c * L + lanes for c in range(CH)]
                for m in range(IPU)]
    zeros16 = jnp.zeros((L,), jnp.float32)

    def zero_acc(acc):
        def zr(r, c2):
            for c in range(CD):
                acc[r, pl.ds(c * L, L)] = zeros16
            return c2
        lax.fori_loop(0, UHW, zr, 0, unroll=2)

    def fire_gathers(u, acc, gsem):
        for k in range(K):
            pltpu.async_copy(table_hbm.at[idx_v.at[u * K + k]], acc, gsem,
                             add=True)

    def wait_gathers(acc, gsem):
        for k in range(K):
            pltpu.make_async_copy(table_hbm.at[idx_v.at[k]], acc, gsem).wait()

    def transpose(acc, accT):
        def per_d(d, c2):
            col = jnp.full((L,), d, dtype=jnp.int32)
            for m in range(IPU):
                for c in range(CH):
                    accT[m * D + d, pl.ds(c * L, L)] = plsc.load_gather(
                        acc, [row_vecs[m][c], col])
            return c2
        lax.fori_loop(0, D, per_d, 0, unroll=2)

    bufs = ((acc0, accT0, gsem0, osem0), (acc1, accT1, gsem1, osem1))

    zero_acc(acc0)
    zero_acc(acc1)
    fire_gathers(0, acc0, gsem0)
    fire_gathers(1, acc1, gsem1)

    def pipe(t, c2):
        for p, (acc, accT, gsem, osem) in enumerate(bufs):
            u = t * 2 + p
            wait_gathers(acc, gsem)

            @pl.when(u >= 2)
            def _():
                pltpu.make_async_copy(
                    accT, out_hbm.at[pl.ds(0, IPU * D)], osem).wait()

            transpose(acc, accT)
            zero_acc(acc)

            @pl.when(u + 2 < UPW)
            def _():
                fire_gathers(u + 2, acc, gsem)

            pltpu.async_copy(
                accT, out_hbm.at[pl.ds((u0 + u) * IPU * D, IPU * D)], osem)
        return c2

    lax.fori_loop(0, UPW // 2, pipe, 0)
    pltpu.make_async_copy(accT0, out_hbm.at[pl.ds(0, IPU * D)], osem0).wait()
    pltpu.make_async_copy(accT1, out_hbm.at[pl.ds(0, IPU * D)], osem1).wait()


def kernel(inputs, table):
    inp2 = (inputs.reshape(B // IPU, IPU, K, HW)
            .transpose(0, 2, 1, 3)
            .reshape(B // IPU * K, IPU * HW))
    mesh = plsc.VectorSubcoreMesh(
        core_axis_name="c", subcore_axis_name="s",
        num_cores=NC, num_subcores=NS,
    )
    out = pl.kernel(
        _sc_body,
        out_type=jax.ShapeDtypeStruct((B * D, HW), jnp.float32),
        mesh=mesh,
        scratch_types=[
            pltpu.VMEM((UPW * K, UHW), jnp.int32),
            pltpu.VMEM((UHW, D), jnp.float32),
            pltpu.VMEM((UHW, D), jnp.float32),
            pltpu.VMEM((IPU * D, HW), jnp.float32),
            pltpu.VMEM((IPU * D, HW), jnp.float32),
            pltpu.SemaphoreType.DMA,
            pltpu.SemaphoreType.DMA,
            pltpu.SemaphoreType.DMA,
            pltpu.SemaphoreType.DMA,
        ],
        compiler_params=pltpu.CompilerParams(needs_layout_passes=False),
    )(inp2, table)
    return out.reshape(B, D, H, W)
